# baseline (speedup 1.0000x reference)
import jax
import jax.numpy as jnp
from jax import lax
from jax.experimental import pallas as pl
from jax.experimental.pallas import tpu as pltpu
from jax.experimental.pallas import tpu_sc as plsc

EMBED = 64
BATCH = 16384
LAM = 1e-05

NC = 2
NS = 16
NW = NC * NS
PW = BATCH // NW
CH = 64
NCH = PW // CH


def _sc_body(h_hbm, r_hbm, p_hbm, n_hbm, ent_hbm, relT_hbm,
             delta_hbm, l2_hbm,
             hidx, ridx, pidx, nidx,
             hbuf, pbuf, nbuf, relv,
             dout, l2v, sem0, sem1):
    wid = lax.axis_index("s") * NC + lax.axis_index("c")
    base = wid * PW
    sems = (sem0, sem1)

    pltpu.sync_copy(relT_hbm, relv)

    def fire(c):
        k = c % 2
        sem = sems[k]
        row0 = base + c * CH
        pltpu.sync_copy(h_hbm.at[pl.ds(row0, CH)], hidx.at[c])
        pltpu.sync_copy(r_hbm.at[pl.ds(row0, CH)], ridx.at[c])
        pltpu.sync_copy(p_hbm.at[pl.ds(row0, CH)], pidx.at[c])
        pltpu.sync_copy(n_hbm.at[pl.ds(row0, CH)], nidx.at[c])

        def body(g, carry):
            sl16 = pl.ds(16 * g, 16)
            hv16 = hidx[c, sl16]
            pv16 = pidx[c, sl16]
            nv16 = nidx[c, sl16]
            for l in range(16):
                i = 16 * g + l
                he = hv16[l]
                pe = pv16[l]
                ne = nv16[l]
                a = i >> 3
                b = i & 7
                pltpu.make_async_copy(ent_hbm.at[he >> 3, he & 7],
                                      hbuf.at[k, a, b], sem).start()
                pltpu.make_async_copy(ent_hbm.at[pe >> 3, pe & 7],
                                      pbuf.at[k, a, b], sem).start()
                pltpu.make_async_copy(ent_hbm.at[ne >> 3, ne & 7],
                                      nbuf.at[k, a, b], sem).start()
            return carry

        lax.fori_loop(0, CH // 16, body, 0)

    def compute(c, l2):
        k = c % 2
        sem = sems[k]
        pltpu.make_async_copy(ent_hbm.at[pl.ds(0, CH // 8)], hbuf.at[k], sem).wait()
        pltpu.make_async_copy(ent_hbm.at[pl.ds(0, CH // 8)], pbuf.at[k], sem).wait()
        pltpu.make_async_copy(ent_hbm.at[pl.ds(0, CH // 8)], nbuf.at[k], sem).wait()

        lane = lax.iota(jnp.int32, 16)

        def group_body(g, l2c):
            re16 = ridx[c, pl.ds(16 * g, 16)]
            for l in range(16):
                i = 16 * g + l
                a = i >> 3
                b = i & 7
                re = jnp.zeros((16,), jnp.int32) + re16[l]
                dl = jnp.zeros((16,), jnp.float32)
                for d in range(EMBED // 16):
                    sl = pl.ds(16 * d, 16)
                    hv = hbuf[k, a, b, sl]
                    rv = plsc.load_gather(relv, [lane + 16 * d, re])
                    pv = pbuf[k, a, b, sl]
                    nv = nbuf[k, a, b, sl]
                    s = hv + rv
                    dp = s - pv
                    dn = s - nv
                    dl = dl + (dp * dp - dn * dn)
                    l2c = l2c + hv * hv + rv * rv + pv * pv + nv * nv
                dout[i >> 3, pl.ds(16 * (i & 7), 16)] = dl
            return l2c

        l2 = lax.fori_loop(0, CH // 16, group_body, l2)
        pltpu.sync_copy(dout, delta_hbm.at[pl.ds(wid * (PW // 8) + c * (CH // 8), CH // 8)])
        return l2

    fire(0)
    l2 = jnp.zeros((16,), jnp.float32)
    for c in range(NCH):
        if c + 1 < NCH:
            fire(c + 1)
        l2 = compute(c, l2)

    l2v[...] = l2
    pltpu.sync_copy(l2v, l2_hbm.at[wid >> 3, pl.ds(16 * (wid & 7), 16)])


_sc_call = pl.kernel(
    _sc_body,
    out_type=[
        jax.ShapeDtypeStruct((BATCH // 8, 128), jnp.float32),
        jax.ShapeDtypeStruct((NW // 8, 128), jnp.float32),
    ],
    mesh=plsc.VectorSubcoreMesh(core_axis_name="c", subcore_axis_name="s"),
    scratch_types=[
        pltpu.VMEM((NCH, CH), jnp.int32),
        pltpu.VMEM((NCH, CH), jnp.int32),
        pltpu.VMEM((NCH, CH), jnp.int32),
        pltpu.VMEM((NCH, CH), jnp.int32),
        pltpu.VMEM((2, CH // 8, 8, EMBED), jnp.float32),
        pltpu.VMEM((2, CH // 8, 8, EMBED), jnp.float32),
        pltpu.VMEM((2, CH // 8, 8, EMBED), jnp.float32),
        pltpu.VMEM((EMBED, 1000), jnp.float32),
        pltpu.VMEM((CH // 8, 128), jnp.float32),
        pltpu.VMEM((16,), jnp.float32),
        pltpu.SemaphoreType.DMA,
        pltpu.SemaphoreType.DMA,
    ],
    compiler_params=pltpu.CompilerParams(needs_layout_passes=False),
)


def _tc_body(x_ref, l2_ref, out_ref):
    x = x_ref[...]
    g = lax.broadcasted_iota(jnp.int32, (128, 8), 0) // 16
    c = lax.broadcasted_iota(jnp.int32, (128, 8), 1)
    m = (g == c).astype(jnp.float32)
    y = lax.dot_general(x, m, (((1,), (0,)), ((), ())),
                        preferred_element_type=jnp.float32)
    sp = jnp.maximum(y, 0.0) + jnp.log1p(jnp.exp(-jnp.abs(y)))
    l2tot = jnp.sum(l2_ref[...])
    loss = jnp.sum(sp) / BATCH + LAM * (l2tot / (2.0 * BATCH))
    out_ref[...] = jnp.full((1, 1), 0.0, jnp.float32) + loss


def kernel(h, r, pos_t, neg_t, entity_embed, relation_embed):
    ent3 = entity_embed.reshape(-1, 8, EMBED)
    relT = relation_embed.T
    delta, l2p = _sc_call(h, r, pos_t, neg_t, ent3, relT)
    out = pl.pallas_call(
        _tc_body,
        out_shape=jax.ShapeDtypeStruct((1, 1), jnp.float32),
    )(delta, l2p)
    return out[0, 0]


# speedup vs baseline: 1.0962x; 1.0962x over previous
import jax
import jax.numpy as jnp
from jax import lax
from jax.experimental import pallas as pl
from jax.experimental.pallas import tpu as pltpu
from jax.experimental.pallas import tpu_sc as plsc

EMBED = 64
BATCH = 16384
LAM = 1e-05

NC = 2
NS = 16
NW = NC * NS
PW = BATCH // NW
CH = 64
NCH = PW // CH


def _sc_body(h_hbm, r_hbm, p_hbm, n_hbm, ent_hbm, rel_hbm,
             delta_hbm, l2_hbm,
             hidx, ridx, pidx, nidx,
             hbuf, rbuf, pbuf, nbuf,
             dout, l2v, sem0, sem1):
    wid = lax.axis_index("s") * NC + lax.axis_index("c")
    base = wid * PW
    sems = (sem0, sem1)

    def fire(c):
        k = c % 2
        sem = sems[k]
        row0 = base + c * CH
        pltpu.sync_copy(h_hbm.at[pl.ds(row0, CH)], hidx.at[c])
        pltpu.sync_copy(r_hbm.at[pl.ds(row0, CH)], ridx.at[c])
        pltpu.sync_copy(p_hbm.at[pl.ds(row0, CH)], pidx.at[c])
        pltpu.sync_copy(n_hbm.at[pl.ds(row0, CH)], nidx.at[c])

        def body(g, carry):
            sl16 = pl.ds(16 * g, 16)
            hv16 = hidx[c, sl16]
            rv16 = ridx[c, sl16]
            pv16 = pidx[c, sl16]
            nv16 = nidx[c, sl16]
            for l in range(16):
                i = 16 * g + l
                he = hv16[l]
                re = rv16[l]
                pe = pv16[l]
                ne = nv16[l]
                a = i >> 3
                b = i & 7
                pltpu.make_async_copy(ent_hbm.at[he >> 3, he & 7],
                                      hbuf.at[k, a, b], sem).start()
                pltpu.make_async_copy(rel_hbm.at[re >> 3, re & 7],
                                      rbuf.at[k, a, b], sem).start()
                pltpu.make_async_copy(ent_hbm.at[pe >> 3, pe & 7],
                                      pbuf.at[k, a, b], sem).start()
                pltpu.make_async_copy(ent_hbm.at[ne >> 3, ne & 7],
                                      nbuf.at[k, a, b], sem).start()
            return carry

        lax.fori_loop(0, CH // 16, body, 0)

    def compute(c, l2):
        k = c % 2
        sem = sems[k]
        pltpu.make_async_copy(ent_hbm.at[pl.ds(0, CH // 8)], hbuf.at[k], sem).wait()
        pltpu.make_async_copy(ent_hbm.at[pl.ds(0, CH // 8)], rbuf.at[k], sem).wait()
        pltpu.make_async_copy(ent_hbm.at[pl.ds(0, CH // 8)], pbuf.at[k], sem).wait()
        pltpu.make_async_copy(ent_hbm.at[pl.ds(0, CH // 8)], nbuf.at[k], sem).wait()

        def row_body(i, l2c):
            a = i >> 3
            b = i & 7
            dl = jnp.zeros((16,), jnp.float32)
            for d in range(EMBED // 16):
                sl = pl.ds(16 * d, 16)
                hv = hbuf[k, a, b, sl]
                rv = rbuf[k, a, b, sl]
                pv = pbuf[k, a, b, sl]
                nv = nbuf[k, a, b, sl]
                s = hv + rv
                dp = s - pv
                dn = s - nv
                dl = dl + (dp * dp - dn * dn)
                l2c = l2c + hv * hv + rv * rv + pv * pv + nv * nv
            dout[i >> 3, pl.ds(16 * (i & 7), 16)] = dl
            return l2c

        l2 = lax.fori_loop(0, CH, row_body, l2)
        pltpu.sync_copy(dout, delta_hbm.at[pl.ds(wid * (PW // 8) + c * (CH // 8), CH // 8)])
        return l2

    fire(0)
    l2 = jnp.zeros((16,), jnp.float32)
    for c in range(NCH):
        if c + 1 < NCH:
            fire(c + 1)
        l2 = compute(c, l2)

    l2v[...] = l2
    pltpu.sync_copy(l2v, l2_hbm.at[wid >> 3, pl.ds(16 * (wid & 7), 16)])


_sc_call = pl.kernel(
    _sc_body,
    out_type=[
        jax.ShapeDtypeStruct((BATCH // 8, 128), jnp.float32),
        jax.ShapeDtypeStruct((NW // 8, 128), jnp.float32),
    ],
    mesh=plsc.VectorSubcoreMesh(core_axis_name="c", subcore_axis_name="s"),
    scratch_types=[
        pltpu.VMEM((NCH, CH), jnp.int32),
        pltpu.VMEM((NCH, CH), jnp.int32),
        pltpu.VMEM((NCH, CH), jnp.int32),
        pltpu.VMEM((NCH, CH), jnp.int32),
        pltpu.VMEM((2, CH // 8, 8, EMBED), jnp.float32),
        pltpu.VMEM((2, CH // 8, 8, EMBED), jnp.float32),
        pltpu.VMEM((2, CH // 8, 8, EMBED), jnp.float32),
        pltpu.VMEM((2, CH // 8, 8, EMBED), jnp.float32),
        pltpu.VMEM((CH // 8, 128), jnp.float32),
        pltpu.VMEM((16,), jnp.float32),
        pltpu.SemaphoreType.DMA,
        pltpu.SemaphoreType.DMA,
    ],
)


def _tc_body(x_ref, l2_ref, out_ref):
    x = x_ref[...]
    g = lax.broadcasted_iota(jnp.int32, (128, 8), 0) // 16
    c = lax.broadcasted_iota(jnp.int32, (128, 8), 1)
    m = (g == c).astype(jnp.float32)
    y = lax.dot_general(x, m, (((1,), (0,)), ((), ())),
                        preferred_element_type=jnp.float32)
    sp = jnp.maximum(y, 0.0) + jnp.log1p(jnp.exp(-jnp.abs(y)))
    l2tot = jnp.sum(l2_ref[...])
    loss = jnp.sum(sp) / BATCH + LAM * (l2tot / (2.0 * BATCH))
    out_ref[...] = jnp.full((1, 1), 0.0, jnp.float32) + loss


def kernel(h, r, pos_t, neg_t, entity_embed, relation_embed):
    ent3 = entity_embed.reshape(-1, 8, EMBED)
    rel3 = relation_embed.reshape(-1, 8, EMBED)
    delta, l2p = _sc_call(h, r, pos_t, neg_t, ent3, rel3)
    out = pl.pallas_call(
        _tc_body,
        out_shape=jax.ShapeDtypeStruct((1, 1), jnp.float32),
    )(delta, l2p)
    return out[0, 0]
